# baseline (speedup 1.0000x reference)
import functools

import jax
import jax.numpy as jnp
from jax import lax
from jax.experimental import pallas as pl
from jax.experimental.pallas import tpu as pltpu
from jax.experimental.pallas import tpu_sc as plsc

N = 10000
D = 128
E = 320000

NC = 2
NS = 16
NW = NC * NS
CK = 128
CW = 80
EP = NW * CW * CK
NP = 10240
RPT = NP // NS
HW = 64
GRID = 8
RB = NP // GRID
GRID_O = 10
RBO = 1024

_mesh = plsc.VectorSubcoreMesh(
    core_axis_name="c", subcore_axis_name="s", num_cores=NC, num_subcores=NS
)
_sc_params = pltpu.CompilerParams(use_tc_tiling_on_sc=False)


@functools.partial(
    pl.kernel,
    out_type=[jax.ShapeDtypeStruct((NP, 16), jnp.float32),
              jax.ShapeDtypeStruct((NP, 16), jnp.float32)],
    mesh=_mesh,
    compiler_params=_sc_params,
    scratch_types=[
        pltpu.VMEM((CW, CK), jnp.int32),
        pltpu.VMEM((CK, 16), jnp.float32),
        pltpu.VMEM((RPT, 16), jnp.float32),
        pltpu.VMEM_SHARED((NP, 16), jnp.float32),
        pltpu.SemaphoreType.DMA,
    ],
)
def _deg_kernel(d_hbm, oh_hbm, z16_hbm, out0_hbm, out1_hbm, dv, oh, zb, acc, sem):
    cid = lax.axis_index("c")
    sid = lax.axis_index("s")
    wid = sid * NC + cid
    base = sid * RPT
    pltpu.sync_copy(d_hbm.at[wid], dv)
    pltpu.sync_copy(oh_hbm, oh)
    pltpu.sync_copy(z16_hbm, zb)
    pltpu.sync_copy(zb, acc.at[pl.ds(base, RPT)])
    plsc.subcore_barrier()

    def _start(j, carry):
        pltpu.async_copy(oh, acc.at[dv.at[j]], sem, add=True)
        return carry

    lax.fori_loop(0, CW, _start, 0)

    def _drain(j, carry):
        pltpu.make_async_copy(oh, acc.at[dv.at[0]], sem).wait()
        return carry

    lax.fori_loop(0, CW, _drain, 0)
    plsc.subcore_barrier()
    pltpu.sync_copy(acc.at[pl.ds(base, RPT)], zb)

    @pl.when(cid == 0)
    def _():
        pltpu.sync_copy(zb, out0_hbm.at[pl.ds(base, RPT)])

    @pl.when(cid != 0)
    def _():
        pltpu.sync_copy(zb, out1_hbm.at[pl.ds(base, RPT)])


def _dis_from_views(d0v, d1v, nrows):
    nv = nrows // 8
    dv = d0v + d1v
    sel = (lax.broadcasted_iota(jnp.int32, (nrows, nv), 0) // 8
           == lax.broadcasted_iota(jnp.int32, (nrows, nv), 1)).astype(jnp.float32)
    rep = jnp.dot(sel, dv, preferred_element_type=jnp.float32)
    lane = (lax.broadcasted_iota(jnp.int32, (nrows, 128), 1)
            == 16 * (lax.broadcasted_iota(jnp.int32, (nrows, 128), 0) % 8))
    deg = jnp.sum(jnp.where(lane, rep, 0.0), axis=1, keepdims=True) + 1.0
    return lax.rsqrt(deg)


def _ka1_body(z_ref, w1_ref, b1_ref, wg_ref, xw_ref):
    h = jnp.dot(z_ref[...], w1_ref[...], preferred_element_type=jnp.float32)
    h = jnp.maximum(h + b1_ref[...], 0.0)
    xw_ref[...] = jnp.dot(h, wg_ref[...], preferred_element_type=jnp.float32)


def _ka1(z_p, W1, b1r, Wg):
    return pl.pallas_call(
        _ka1_body,
        grid=(GRID,),
        in_specs=[
            pl.BlockSpec((RB, D), lambda i: (i, 0)),
            pl.BlockSpec((D, D), lambda i: (0, 0)),
            pl.BlockSpec((1, D), lambda i: (0, 0)),
            pl.BlockSpec((D, D), lambda i: (0, 0)),
        ],
        out_specs=pl.BlockSpec((RB, D), lambda i: (i, 0)),
        out_shape=jax.ShapeDtypeStruct((NP, D), jnp.float32),
    )(z_p, W1, b1r, Wg)


def _ka2_body(xw_ref, d0_ref, d1_ref, y_ref):
    y_ref[...] = xw_ref[...] * _dis_from_views(d0_ref[...], d1_ref[...], RB)


def _ka2(xw, deg0, deg1):
    return pl.pallas_call(
        _ka2_body,
        grid=(GRID,),
        in_specs=[
            pl.BlockSpec((RB, D), lambda i: (i, 0)),
            pl.BlockSpec((RB // 8, D), lambda i: (i, 0)),
            pl.BlockSpec((RB // 8, D), lambda i: (i, 0)),
        ],
        out_specs=pl.BlockSpec((RB, D), lambda i: (i, 0)),
        out_shape=jax.ShapeDtypeStruct((NP, D), jnp.float32),
    )(xw, deg0, deg1)


@functools.partial(
    pl.kernel,
    out_type=jax.ShapeDtypeStruct((NP, D), jnp.float32),
    mesh=_mesh,
    compiler_params=_sc_params,
    scratch_types=[
        pltpu.VMEM((CW, CK), jnp.int32),
        pltpu.VMEM((CW, CK), jnp.int32),
        pltpu.VMEM((CK, HW), jnp.float32),
        pltpu.VMEM((CK, HW), jnp.float32),
        pltpu.VMEM((CK, HW), jnp.float32),
        pltpu.VMEM((CK, HW), jnp.float32),
        pltpu.VMEM((CK, HW), jnp.float32),
        pltpu.VMEM((CK, HW), jnp.float32),
        pltpu.VMEM((CK, HW), jnp.float32),
        pltpu.VMEM((CK, HW), jnp.float32),
        pltpu.VMEM_SHARED((NP, HW), jnp.float32),
        [pltpu.SemaphoreType.DMA] * 8,
        [pltpu.SemaphoreType.DMA] * 8,
    ],
)
def _seg_kernel(y2_hbm, s_hbm, d_hbm, zslab_hbm, p_hbm,
                sv, dv, rb0, rb1, rb2, rb3, rb4, rb5, rb6, rb7,
                acc, gs, sse):
    cid = lax.axis_index("c")
    sid = lax.axis_index("s")
    base = sid * RPT
    off = cid * HW

    pltpu.sync_copy(zslab_hbm, acc.at[pl.ds(base, RPT)])
    plsc.subcore_barrier()

    def _mkidx(r, carry):
        for c8 in range(CK // 16):
            sl = pl.ds(16 * c8, 16)
            sv[r, sl] = sv[r, sl] * 2 + cid
        return carry

    rbs = (rb0, rb1, rb2, rb3, rb4, rb5, rb6, rb7)
    NB = 8

    for half in range(2):
        wrow = sid * 2 + half
        pltpu.sync_copy(s_hbm.at[wrow], sv)
        pltpu.sync_copy(d_hbm.at[wrow], dv)
        lax.fori_loop(0, CW, _mkidx, 0)

        for b in range(NB // 2):
            pltpu.async_copy(y2_hbm.at[sv.at[b]], rbs[b], gs[b])
        for j in range(NB // 2):
            pltpu.make_async_copy(y2_hbm.at[sv.at[j]], rbs[j], gs[j]).wait()
            pltpu.async_copy(rbs[j], acc.at[dv.at[j]], sse[j], add=True)
            pltpu.async_copy(y2_hbm.at[sv.at[j + 4]], rbs[j + 4], gs[j + 4])

        def _body(t, carry):
            for b8 in range(NB):
                j = 4 + NB * t + b8
                bb = (4 + b8) % NB
                br = b8
                pltpu.make_async_copy(y2_hbm.at[sv.at[j]], rbs[bb], gs[bb]).wait()
                pltpu.async_copy(rbs[bb], acc.at[dv.at[j]], sse[bb], add=True)
                pltpu.make_async_copy(rbs[br], acc.at[dv.at[0]], sse[br]).wait()
                pltpu.async_copy(y2_hbm.at[sv.at[j + 4]], rbs[br], gs[br])
            return carry

        lax.fori_loop(0, (CW - 8) // NB, _body, 0)
        for j in (CW - 4, CW - 3, CW - 2, CW - 1):
            bb = j % NB
            pltpu.make_async_copy(y2_hbm.at[sv.at[j]], rbs[bb], gs[bb]).wait()
            pltpu.async_copy(rbs[bb], acc.at[dv.at[j]], sse[bb], add=True)
        for b in range(NB):
            pltpu.make_async_copy(rbs[b], acc.at[dv.at[0]], sse[b]).wait()

    plsc.subcore_barrier()
    pltpu.sync_copy(acc.at[pl.ds(base, RPT)],
                    p_hbm.at[pl.ds(base, RPT), pl.ds(off, HW)])


def _kc_body(p_ref, y_ref, d0_ref, d1_ref, bg_ref, w2_ref, b2_ref, o_ref):
    dis = _dis_from_views(d0_ref[...], d1_ref[...], RBO)
    h = jnp.maximum((p_ref[...] + y_ref[...]) * dis + bg_ref[...], 0.0)
    o = jnp.dot(h, w2_ref[...], preferred_element_type=jnp.float32)
    o = o + b2_ref[...]
    col = lax.broadcasted_iota(jnp.int32, (RBO, D), 1)
    o_ref[...] = jnp.where(col == 0, jax.nn.sigmoid(o), o)


def _kc(p, y, deg0, deg1, bgr, W2, b2r):
    return pl.pallas_call(
        _kc_body,
        grid=(GRID_O,),
        in_specs=[
            pl.BlockSpec((RBO, D), lambda i: (i, 0)),
            pl.BlockSpec((RBO, D), lambda i: (i, 0)),
            pl.BlockSpec((RBO // 8, D), lambda i: (i, 0)),
            pl.BlockSpec((RBO // 8, D), lambda i: (i, 0)),
            pl.BlockSpec((1, D), lambda i: (0, 0)),
            pl.BlockSpec((D, D), lambda i: (0, 0)),
            pl.BlockSpec((1, D), lambda i: (0, 0)),
        ],
        out_specs=pl.BlockSpec((RBO, D), lambda i: (i, 0)),
        out_shape=jax.ShapeDtypeStruct((N, D), jnp.float32),
    )(p, y, deg0, deg1, bgr, W2, b2r)


@jax.jit
def kernel(z, W1, b1, Wg, bg, W2, b2, edge_index):
    z_p = jnp.pad(z, ((0, NP - N), (0, 0)))
    b1r = b1.reshape(1, D)
    bgr = bg.reshape(1, D)
    b2r = b2.reshape(1, D)

    npad = EP - E
    pad_idx = (N + (jnp.arange(npad, dtype=jnp.int32) % (NP - N))).astype(jnp.int32)
    s_r = jnp.concatenate([edge_index[0], pad_idx]).reshape(NW, CW, CK)
    d_r = jnp.concatenate([edge_index[1], pad_idx]).reshape(NW, CW, CK)

    onehot = jnp.zeros((CK, 16), jnp.float32).at[:, 0].set(1.0)
    zeros16 = jnp.zeros((RPT, 16), jnp.float32)
    zslab = jnp.zeros((RPT, HW), jnp.float32)

    xw = _ka1(z_p, W1, b1r, Wg)
    deg0, deg1 = _deg_kernel(d_r, onehot, zeros16)
    deg0v = deg0.reshape(NP // 8, D)
    deg1v = deg1.reshape(NP // 8, D)
    y = _ka2(xw, deg0v, deg1v)
    y2 = y.reshape(2 * NP, HW)
    p = _seg_kernel(y2, s_r, d_r, zslab)
    return _kc(p, y, deg0v, deg1v, bgr, W2, b2r)


# speedup vs baseline: 1.0025x; 1.0025x over previous
import functools

import jax
import jax.numpy as jnp
from jax import lax
from jax.experimental import pallas as pl
from jax.experimental.pallas import tpu as pltpu
from jax.experimental.pallas import tpu_sc as plsc

N = 10000
D = 128
E = 320000

NC = 2
NS = 16
NW = NC * NS
CK = 128
CW = 80
EP = NW * CW * CK
NP = 10240
RPT = NP // NS
HW = 64
GRID = 8
RB = NP // GRID
GRID_O = 10
RBO = 1024

_mesh = plsc.VectorSubcoreMesh(
    core_axis_name="c", subcore_axis_name="s", num_cores=NC, num_subcores=NS
)
_sc_params = pltpu.CompilerParams(use_tc_tiling_on_sc=False)


@functools.partial(
    pl.kernel,
    out_type=[jax.ShapeDtypeStruct((NP, 16), jnp.float32),
              jax.ShapeDtypeStruct((NP, 16), jnp.float32)],
    mesh=_mesh,
    compiler_params=_sc_params,
    scratch_types=[
        pltpu.VMEM((CW, CK), jnp.int32),
        pltpu.VMEM((CK, 16), jnp.float32),
        pltpu.VMEM((RPT, 16), jnp.float32),
        pltpu.VMEM_SHARED((NP, 16), jnp.float32),
        pltpu.SemaphoreType.DMA,
    ],
)
def _deg_kernel(d_hbm, oh_hbm, z16_hbm, out0_hbm, out1_hbm, dv, oh, zb, acc, sem):
    cid = lax.axis_index("c")
    sid = lax.axis_index("s")
    wid = sid * NC + cid
    base = sid * RPT
    pltpu.sync_copy(d_hbm.at[wid], dv)
    pltpu.sync_copy(oh_hbm, oh)
    pltpu.sync_copy(z16_hbm, zb)
    pltpu.sync_copy(zb, acc.at[pl.ds(base, RPT)])
    plsc.subcore_barrier()

    def _start(j, carry):
        pltpu.async_copy(oh, acc.at[dv.at[j]], sem, add=True)
        return carry

    lax.fori_loop(0, CW, _start, 0)

    def _drain(j, carry):
        pltpu.make_async_copy(oh, acc.at[dv.at[0]], sem).wait()
        return carry

    lax.fori_loop(0, CW, _drain, 0)
    plsc.subcore_barrier()
    pltpu.sync_copy(acc.at[pl.ds(base, RPT)], zb)

    @pl.when(cid == 0)
    def _():
        pltpu.sync_copy(zb, out0_hbm.at[pl.ds(base, RPT)])

    @pl.when(cid != 0)
    def _():
        pltpu.sync_copy(zb, out1_hbm.at[pl.ds(base, RPT)])


def _dis_from_views(d0v, d1v, nrows):
    nv = nrows // 8
    dv = d0v + d1v
    sel = (lax.broadcasted_iota(jnp.int32, (nrows, nv), 0) // 8
           == lax.broadcasted_iota(jnp.int32, (nrows, nv), 1)).astype(jnp.float32)
    rep = jnp.dot(sel, dv, preferred_element_type=jnp.float32)
    lane = (lax.broadcasted_iota(jnp.int32, (nrows, 128), 1)
            == 16 * (lax.broadcasted_iota(jnp.int32, (nrows, 128), 0) % 8))
    deg = jnp.sum(jnp.where(lane, rep, 0.0), axis=1, keepdims=True) + 1.0
    return lax.rsqrt(deg)


def _ka_body(z_ref, w1_ref, b1_ref, wg_ref, d0_ref, d1_ref, y_ref):
    h = jnp.dot(z_ref[...], w1_ref[...], preferred_element_type=jnp.float32)
    h = jnp.maximum(h + b1_ref[...], 0.0)
    xw = jnp.dot(h, wg_ref[...], preferred_element_type=jnp.float32)
    y_ref[...] = xw * _dis_from_views(d0_ref[...], d1_ref[...], RB)


def _ka(z_p, W1, b1r, Wg, deg0, deg1):
    return pl.pallas_call(
        _ka_body,
        grid=(GRID,),
        in_specs=[
            pl.BlockSpec((RB, D), lambda i: (i, 0)),
            pl.BlockSpec((D, D), lambda i: (0, 0)),
            pl.BlockSpec((1, D), lambda i: (0, 0)),
            pl.BlockSpec((D, D), lambda i: (0, 0)),
            pl.BlockSpec((RB // 8, D), lambda i: (i, 0)),
            pl.BlockSpec((RB // 8, D), lambda i: (i, 0)),
        ],
        out_specs=pl.BlockSpec((RB, D), lambda i: (i, 0)),
        out_shape=jax.ShapeDtypeStruct((NP, D), jnp.float32),
    )(z_p, W1, b1r, Wg, deg0, deg1)


@functools.partial(
    pl.kernel,
    out_type=jax.ShapeDtypeStruct((NP, D), jnp.float32),
    mesh=_mesh,
    compiler_params=_sc_params,
    scratch_types=[
        pltpu.VMEM((CW, CK), jnp.int32),
        pltpu.VMEM((CW, CK), jnp.int32),
        pltpu.VMEM((CK, HW), jnp.float32),
        pltpu.VMEM((CK, HW), jnp.float32),
        pltpu.VMEM((CK, HW), jnp.float32),
        pltpu.VMEM((CK, HW), jnp.float32),
        pltpu.VMEM((CK, HW), jnp.float32),
        pltpu.VMEM((CK, HW), jnp.float32),
        pltpu.VMEM((CK, HW), jnp.float32),
        pltpu.VMEM((CK, HW), jnp.float32),
        pltpu.VMEM_SHARED((NP, HW), jnp.float32),
        [pltpu.SemaphoreType.DMA] * 8,
        [pltpu.SemaphoreType.DMA] * 8,
    ],
)
def _seg_kernel(y2_hbm, s_hbm, d_hbm, zslab_hbm, p_hbm,
                sv, dv, rb0, rb1, rb2, rb3, rb4, rb5, rb6, rb7,
                acc, gs, sse):
    cid = lax.axis_index("c")
    sid = lax.axis_index("s")
    base = sid * RPT
    off = cid * HW

    pltpu.sync_copy(zslab_hbm, acc.at[pl.ds(base, RPT)])
    plsc.subcore_barrier()

    def _mkidx(r, carry):
        for c8 in range(CK // 16):
            sl = pl.ds(16 * c8, 16)
            sv[r, sl] = sv[r, sl] * 2 + cid
        return carry

    rbs = (rb0, rb1, rb2, rb3, rb4, rb5, rb6, rb7)
    NB = 8

    for half in range(2):
        wrow = sid * 2 + half
        pltpu.sync_copy(s_hbm.at[wrow], sv)
        pltpu.sync_copy(d_hbm.at[wrow], dv)
        lax.fori_loop(0, CW, _mkidx, 0)

        for b in range(NB // 2):
            pltpu.async_copy(y2_hbm.at[sv.at[b]], rbs[b], gs[b])
        for j in range(NB // 2):
            pltpu.make_async_copy(y2_hbm.at[sv.at[j]], rbs[j], gs[j]).wait()
            pltpu.async_copy(rbs[j], acc.at[dv.at[j]], sse[j], add=True)
            pltpu.async_copy(y2_hbm.at[sv.at[j + 4]], rbs[j + 4], gs[j + 4])

        def _body(t, carry):
            for b8 in range(NB):
                j = 4 + NB * t + b8
                bb = (4 + b8) % NB
                br = b8
                pltpu.make_async_copy(y2_hbm.at[sv.at[j]], rbs[bb], gs[bb]).wait()
                pltpu.async_copy(rbs[bb], acc.at[dv.at[j]], sse[bb], add=True)
                pltpu.make_async_copy(rbs[br], acc.at[dv.at[0]], sse[br]).wait()
                pltpu.async_copy(y2_hbm.at[sv.at[j + 4]], rbs[br], gs[br])
            return carry

        lax.fori_loop(0, (CW - 8) // NB, _body, 0)
        for j in (CW - 4, CW - 3, CW - 2, CW - 1):
            bb = j % NB
            pltpu.make_async_copy(y2_hbm.at[sv.at[j]], rbs[bb], gs[bb]).wait()
            pltpu.async_copy(rbs[bb], acc.at[dv.at[j]], sse[bb], add=True)
        for b in range(NB):
            pltpu.make_async_copy(rbs[b], acc.at[dv.at[0]], sse[b]).wait()

    plsc.subcore_barrier()
    pltpu.sync_copy(acc.at[pl.ds(base, RPT)],
                    p_hbm.at[pl.ds(base, RPT), pl.ds(off, HW)])


def _kc_body(p_ref, y_ref, d0_ref, d1_ref, bg_ref, w2_ref, b2_ref, o_ref):
    dis = _dis_from_views(d0_ref[...], d1_ref[...], RBO)
    h = jnp.maximum((p_ref[...] + y_ref[...]) * dis + bg_ref[...], 0.0)
    o = jnp.dot(h, w2_ref[...], preferred_element_type=jnp.float32)
    o = o + b2_ref[...]
    col = lax.broadcasted_iota(jnp.int32, (RBO, D), 1)
    o_ref[...] = jnp.where(col == 0, jax.nn.sigmoid(o), o)


def _kc(p, y, deg0, deg1, bgr, W2, b2r):
    return pl.pallas_call(
        _kc_body,
        grid=(GRID_O,),
        in_specs=[
            pl.BlockSpec((RBO, D), lambda i: (i, 0)),
            pl.BlockSpec((RBO, D), lambda i: (i, 0)),
            pl.BlockSpec((RBO // 8, D), lambda i: (i, 0)),
            pl.BlockSpec((RBO // 8, D), lambda i: (i, 0)),
            pl.BlockSpec((1, D), lambda i: (0, 0)),
            pl.BlockSpec((D, D), lambda i: (0, 0)),
            pl.BlockSpec((1, D), lambda i: (0, 0)),
        ],
        out_specs=pl.BlockSpec((RBO, D), lambda i: (i, 0)),
        out_shape=jax.ShapeDtypeStruct((N, D), jnp.float32),
    )(p, y, deg0, deg1, bgr, W2, b2r)


@jax.jit
def kernel(z, W1, b1, Wg, bg, W2, b2, edge_index):
    z_p = jnp.pad(z, ((0, NP - N), (0, 0)))
    b1r = b1.reshape(1, D)
    bgr = bg.reshape(1, D)
    b2r = b2.reshape(1, D)

    npad = EP - E
    pad_idx = (N + (jnp.arange(npad, dtype=jnp.int32) % (NP - N))).astype(jnp.int32)
    s_r = jnp.concatenate([edge_index[0], pad_idx]).reshape(NW, CW, CK)
    d_r = jnp.concatenate([edge_index[1], pad_idx]).reshape(NW, CW, CK)

    onehot = jnp.zeros((CK, 16), jnp.float32).at[:, 0].set(1.0)
    zeros16 = jnp.zeros((RPT, 16), jnp.float32)
    zslab = jnp.zeros((RPT, HW), jnp.float32)

    deg0, deg1 = _deg_kernel(d_r, onehot, zeros16)
    deg0v = deg0.reshape(NP // 8, D)
    deg1v = deg1.reshape(NP // 8, D)
    y = _ka(z_p, W1, b1r, Wg, deg0v, deg1v)
    y2 = y.reshape(2 * NP, HW)
    p = _seg_kernel(y2, s_r, d_r, zslab)
    return _kc(p, y, deg0v, deg1v, bgr, W2, b2r)


# speedup vs baseline: 1.0030x; 1.0005x over previous
import functools

import jax
import jax.numpy as jnp
from jax import lax
from jax.experimental import pallas as pl
from jax.experimental.pallas import tpu as pltpu
from jax.experimental.pallas import tpu_sc as plsc

N = 10000
D = 128
E = 320000

NC = 2
NS = 16
NW = NC * NS
CK = 128
CW = 80
EP = NW * CW * CK
NP = 10240
RPT = NP // NS
HW = 64
GRID = 8
RB = NP // GRID
GRID_O = 10
RBO = 1024

_mesh = plsc.VectorSubcoreMesh(
    core_axis_name="c", subcore_axis_name="s", num_cores=NC, num_subcores=NS
)
_sc_params = pltpu.CompilerParams(use_tc_tiling_on_sc=False)


@functools.partial(
    pl.kernel,
    out_type=[jax.ShapeDtypeStruct((NP, 16), jnp.float32),
              jax.ShapeDtypeStruct((NP, 16), jnp.float32)],
    mesh=_mesh,
    compiler_params=_sc_params,
    scratch_types=[
        pltpu.VMEM((CW, CK), jnp.int32),
        pltpu.VMEM((CK, 16), jnp.float32),
        pltpu.VMEM((RPT, 16), jnp.float32),
        pltpu.VMEM_SHARED((NP, 16), jnp.float32),
        pltpu.SemaphoreType.DMA,
    ],
)
def _deg_kernel(d_hbm, oh_hbm, z16_hbm, out0_hbm, out1_hbm, dv, oh, zb, acc, sem):
    cid = lax.axis_index("c")
    sid = lax.axis_index("s")
    wid = sid * NC + cid
    base = sid * RPT
    pltpu.sync_copy(d_hbm.at[wid], dv)
    pltpu.sync_copy(oh_hbm, oh)
    pltpu.sync_copy(z16_hbm, zb)
    pltpu.sync_copy(zb, acc.at[pl.ds(base, RPT)])
    plsc.subcore_barrier()

    def _start(j, carry):
        pltpu.async_copy(oh, acc.at[dv.at[j]], sem, add=True)
        return carry

    lax.fori_loop(0, 16, _start, 0)

    def _slide(j, carry):
        pltpu.make_async_copy(oh, acc.at[dv.at[0]], sem).wait()
        pltpu.async_copy(oh, acc.at[dv.at[j + 16]], sem, add=True)
        return carry

    lax.fori_loop(0, CW - 16, _slide, 0)

    def _drain(j, carry):
        pltpu.make_async_copy(oh, acc.at[dv.at[0]], sem).wait()
        return carry

    lax.fori_loop(0, 16, _drain, 0)
    plsc.subcore_barrier()
    pltpu.sync_copy(acc.at[pl.ds(base, RPT)], zb)

    @pl.when(cid == 0)
    def _():
        pltpu.sync_copy(zb, out0_hbm.at[pl.ds(base, RPT)])

    @pl.when(cid != 0)
    def _():
        pltpu.sync_copy(zb, out1_hbm.at[pl.ds(base, RPT)])


def _dis_from_views(d0v, d1v, nrows):
    nv = nrows // 8
    dv = d0v + d1v
    sel = (lax.broadcasted_iota(jnp.int32, (nrows, nv), 0) // 8
           == lax.broadcasted_iota(jnp.int32, (nrows, nv), 1)).astype(jnp.float32)
    rep = jnp.dot(sel, dv, preferred_element_type=jnp.float32)
    lane = (lax.broadcasted_iota(jnp.int32, (nrows, 128), 1)
            == 16 * (lax.broadcasted_iota(jnp.int32, (nrows, 128), 0) % 8))
    deg = jnp.sum(jnp.where(lane, rep, 0.0), axis=1, keepdims=True) + 1.0
    return lax.rsqrt(deg)


def _ka_body(z_ref, w1_ref, b1_ref, wg_ref, d0_ref, d1_ref, y_ref):
    h = jnp.dot(z_ref[...], w1_ref[...], preferred_element_type=jnp.float32)
    h = jnp.maximum(h + b1_ref[...], 0.0)
    xw = jnp.dot(h, wg_ref[...], preferred_element_type=jnp.float32)
    y_ref[...] = xw * _dis_from_views(d0_ref[...], d1_ref[...], RB)


def _ka(z_p, W1, b1r, Wg, deg0, deg1):
    return pl.pallas_call(
        _ka_body,
        grid=(GRID,),
        in_specs=[
            pl.BlockSpec((RB, D), lambda i: (i, 0)),
            pl.BlockSpec((D, D), lambda i: (0, 0)),
            pl.BlockSpec((1, D), lambda i: (0, 0)),
            pl.BlockSpec((D, D), lambda i: (0, 0)),
            pl.BlockSpec((RB // 8, D), lambda i: (i, 0)),
            pl.BlockSpec((RB // 8, D), lambda i: (i, 0)),
        ],
        out_specs=pl.BlockSpec((RB, D), lambda i: (i, 0)),
        out_shape=jax.ShapeDtypeStruct((NP, D), jnp.float32),
    )(z_p, W1, b1r, Wg, deg0, deg1)


@functools.partial(
    pl.kernel,
    out_type=jax.ShapeDtypeStruct((NP, D), jnp.float32),
    mesh=_mesh,
    compiler_params=_sc_params,
    scratch_types=[
        pltpu.VMEM((CW, CK), jnp.int32),
        pltpu.VMEM((CW, CK), jnp.int32),
        pltpu.VMEM((CK, HW), jnp.float32),
        pltpu.VMEM((CK, HW), jnp.float32),
        pltpu.VMEM((CK, HW), jnp.float32),
        pltpu.VMEM((CK, HW), jnp.float32),
        pltpu.VMEM((CK, HW), jnp.float32),
        pltpu.VMEM((CK, HW), jnp.float32),
        pltpu.VMEM((CK, HW), jnp.float32),
        pltpu.VMEM((CK, HW), jnp.float32),
        pltpu.VMEM_SHARED((NP, HW), jnp.float32),
        [pltpu.SemaphoreType.DMA] * 8,
        [pltpu.SemaphoreType.DMA] * 8,
    ],
)
def _seg_kernel(y2_hbm, s_hbm, d_hbm, zslab_hbm, p_hbm,
                sv, dv, rb0, rb1, rb2, rb3, rb4, rb5, rb6, rb7,
                acc, gs, sse):
    cid = lax.axis_index("c")
    sid = lax.axis_index("s")
    base = sid * RPT
    off = cid * HW

    pltpu.sync_copy(zslab_hbm, acc.at[pl.ds(base, RPT)])
    plsc.subcore_barrier()

    def _mkidx(r, carry):
        for c8 in range(CK // 16):
            sl = pl.ds(16 * c8, 16)
            sv[r, sl] = sv[r, sl] * 2 + cid
        return carry

    rbs = (rb0, rb1, rb2, rb3, rb4, rb5, rb6, rb7)
    NB = 8

    for half in range(2):
        wrow = sid * 2 + half
        pltpu.sync_copy(s_hbm.at[wrow], sv)
        pltpu.sync_copy(d_hbm.at[wrow], dv)
        lax.fori_loop(0, CW, _mkidx, 0)

        for b in range(NB // 2):
            pltpu.async_copy(y2_hbm.at[sv.at[b]], rbs[b], gs[b])
        for j in range(NB // 2):
            pltpu.make_async_copy(y2_hbm.at[sv.at[j]], rbs[j], gs[j]).wait()
            pltpu.async_copy(rbs[j], acc.at[dv.at[j]], sse[j], add=True)
            pltpu.async_copy(y2_hbm.at[sv.at[j + 4]], rbs[j + 4], gs[j + 4])

        def _body(t, carry):
            for b8 in range(NB):
                j = 4 + NB * t + b8
                bb = (4 + b8) % NB
                br = b8
                pltpu.make_async_copy(y2_hbm.at[sv.at[j]], rbs[bb], gs[bb]).wait()
                pltpu.async_copy(rbs[bb], acc.at[dv.at[j]], sse[bb], add=True)
                pltpu.make_async_copy(rbs[br], acc.at[dv.at[0]], sse[br]).wait()
                pltpu.async_copy(y2_hbm.at[sv.at[j + 4]], rbs[br], gs[br])
            return carry

        lax.fori_loop(0, (CW - 8) // NB, _body, 0)
        for j in (CW - 4, CW - 3, CW - 2, CW - 1):
            bb = j % NB
            pltpu.make_async_copy(y2_hbm.at[sv.at[j]], rbs[bb], gs[bb]).wait()
            pltpu.async_copy(rbs[bb], acc.at[dv.at[j]], sse[bb], add=True)
        for b in range(NB):
            pltpu.make_async_copy(rbs[b], acc.at[dv.at[0]], sse[b]).wait()

    plsc.subcore_barrier()
    pltpu.sync_copy(acc.at[pl.ds(base, RPT)],
                    p_hbm.at[pl.ds(base, RPT), pl.ds(off, HW)])


def _kc_body(p_ref, y_ref, d0_ref, d1_ref, bg_ref, w2_ref, b2_ref, o_ref):
    dis = _dis_from_views(d0_ref[...], d1_ref[...], RBO)
    h = jnp.maximum((p_ref[...] + y_ref[...]) * dis + bg_ref[...], 0.0)
    o = jnp.dot(h, w2_ref[...], preferred_element_type=jnp.float32)
    o = o + b2_ref[...]
    col = lax.broadcasted_iota(jnp.int32, (RBO, D), 1)
    o_ref[...] = jnp.where(col == 0, jax.nn.sigmoid(o), o)


def _kc(p, y, deg0, deg1, bgr, W2, b2r):
    return pl.pallas_call(
        _kc_body,
        grid=(GRID_O,),
        in_specs=[
            pl.BlockSpec((RBO, D), lambda i: (i, 0)),
            pl.BlockSpec((RBO, D), lambda i: (i, 0)),
            pl.BlockSpec((RBO // 8, D), lambda i: (i, 0)),
            pl.BlockSpec((RBO // 8, D), lambda i: (i, 0)),
            pl.BlockSpec((1, D), lambda i: (0, 0)),
            pl.BlockSpec((D, D), lambda i: (0, 0)),
            pl.BlockSpec((1, D), lambda i: (0, 0)),
        ],
        out_specs=pl.BlockSpec((RBO, D), lambda i: (i, 0)),
        out_shape=jax.ShapeDtypeStruct((N, D), jnp.float32),
    )(p, y, deg0, deg1, bgr, W2, b2r)


@jax.jit
def kernel(z, W1, b1, Wg, bg, W2, b2, edge_index):
    z_p = jnp.pad(z, ((0, NP - N), (0, 0)))
    b1r = b1.reshape(1, D)
    bgr = bg.reshape(1, D)
    b2r = b2.reshape(1, D)

    npad = EP - E
    pad_idx = (N + (jnp.arange(npad, dtype=jnp.int32) % (NP - N))).astype(jnp.int32)
    s_r = jnp.concatenate([edge_index[0], pad_idx]).reshape(NW, CW, CK)
    d_r = jnp.concatenate([edge_index[1], pad_idx]).reshape(NW, CW, CK)

    onehot = jnp.zeros((CK, 16), jnp.float32).at[:, 0].set(1.0)
    zeros16 = jnp.zeros((RPT, 16), jnp.float32)
    zslab = jnp.zeros((RPT, HW), jnp.float32)

    deg0, deg1 = _deg_kernel(d_r, onehot, zeros16)
    deg0v = deg0.reshape(NP // 8, D)
    deg1v = deg1.reshape(NP // 8, D)
    y = _ka(z_p, W1, b1r, Wg, deg0v, deg1v)
    y2 = y.reshape(2 * NP, HW)
    p = _seg_kernel(y2, s_r, d_r, zslab)
    return _kc(p, y, deg0v, deg1v, bgr, W2, b2r)
